# Initial kernel scaffold; baseline (speedup 1.0000x reference)
import jax
import jax.numpy as jnp
from jax.experimental import pallas as pl


def kernel(unknown, known, unknow_feats, known_feats, W0, b0, gamma0, beta0, W1, b1, gamma1, beta1):
    raise NotImplementedError("write your pallas kernel here")


# speedup vs baseline: 394.2926x; 394.2926x over previous
import functools

import jax
import jax.numpy as jnp
from jax.experimental import pallas as pl
from jax.experimental.pallas import tpu as pltpu

_TN = 256
_MASK = -2048
_TOP = 2147483647


def _body(u_ref, k_ref, kf_ref, uf_ref, w0a_ref, w0b_ref, w1_ref,
          b0_ref, g0_ref, bt0_ref, b1_ref, g1_ref, bt1_ref,
          out_ref, y0_scr, y1_scr, st_scr, *, bn):
    p = pl.program_id(0)
    t = pl.program_id(1)
    ninv = 1.0 / float(bn)

    @pl.when(p == 0)
    def _phase0():
        u = u_ref[0]
        kk = k_ref[0]
        dt = jnp.dot(kk, u, preferred_element_type=jnp.float32)
        u2 = jnp.sum(u * u, axis=0, keepdims=True)
        k2 = jnp.sum(kk * kk, axis=1, keepdims=True)
        d2 = (u2 - 2.0 * dt) + k2
        ib = jax.lax.bitcast_convert_type(d2, jnp.int32)
        mk = ib ^ (jnp.right_shift(ib, 31) & jnp.int32(_TOP))
        big = jnp.int32(_TOP)
        m1 = jnp.min(mk, axis=0, keepdims=True)
        kx1 = jnp.where(mk <= m1, big, mk)
        m2 = jnp.min(kx1, axis=0, keepdims=True)
        kx2 = jnp.where(kx1 <= m2, big, kx1)
        m3 = jnp.min(kx2, axis=0, keepdims=True)

        tod = lambda v: jax.lax.bitcast_convert_type(
            v ^ (jnp.right_shift(v, 31) & jnp.int32(_TOP)), jnp.float32)
        r1 = 1.0 / (tod(m1) + 1e-8)
        r2 = 1.0 / (tod(m2) + 1e-8)
        r3 = 1.0 / (tod(m3) + 1e-8)
        inv_norm = 1.0 / (r1 + r2 + r3)
        r_elem = 1.0 / (d2 + 1e-8)
        sel = jnp.where(mk <= m3, r_elem * inv_norm, 0.0)

        interp = jnp.dot(kf_ref[0], sel, preferred_element_type=jnp.float32)
        y0 = (jnp.dot(w0a_ref[...], interp, preferred_element_type=jnp.float32)
              + jnp.dot(w0b_ref[...], uf_ref[0], preferred_element_type=jnp.float32)
              + b0_ref[...])
        y0_scr[t] = y0

        @pl.when(t == 0)
        def _init():
            st_scr[...] = jnp.zeros_like(st_scr)

        st_scr[:, 0:1] += jnp.sum(y0, axis=1, keepdims=True)
        st_scr[:, 1:2] += jnp.sum(y0 * y0, axis=1, keepdims=True)

    @pl.when(p == 1)
    def _phase1():
        y0 = y0_scr[t]
        mean = st_scr[:, 0:1] * ninv
        var = st_scr[:, 1:2] * ninv - mean * mean
        sc = g0_ref[...] * jax.lax.rsqrt(var + 1e-5)
        sh = bt0_ref[...] - mean * sc
        h = jnp.maximum(y0 * sc + sh, 0.0)
        y1 = jnp.dot(w1_ref[...], h, preferred_element_type=jnp.float32) + b1_ref[...]
        y1_scr[t] = y1

        @pl.when(t == 0)
        def _init():
            st_scr[:, 2:4] = jnp.zeros_like(st_scr[:, 2:4])

        st_scr[:, 2:3] += jnp.sum(y1, axis=1, keepdims=True)
        st_scr[:, 3:4] += jnp.sum(y1 * y1, axis=1, keepdims=True)

    @pl.when(p == 2)
    def _phase2():
        y1 = y1_scr[t]
        mean = st_scr[:, 2:3] * ninv
        var = st_scr[:, 3:4] * ninv - mean * mean
        sc = g1_ref[...] * jax.lax.rsqrt(var + 1e-5)
        sh = bt1_ref[...] - mean * sc
        out_ref[0] = jnp.maximum(y1 * sc + sh, 0.0)


@jax.jit
def kernel(unknown, known, unknow_feats, known_feats,
           W0, b0, gamma0, beta0, W1, b1, gamma1, beta1):
    B, N, _ = unknown.shape
    M = known.shape[1]
    C1 = unknow_feats.shape[1]
    C2 = known_feats.shape[1]
    K0 = W0.shape[0]
    K1 = W1.shape[0]
    tn = _TN
    tpb = N // tn
    nt = B * tpb
    bn = B * N

    uT8 = jnp.concatenate(
        [jnp.swapaxes(unknown, 1, 2),
         jnp.zeros((B, 5, N), unknown.dtype)], axis=1)
    k8 = jnp.concatenate(
        [known, jnp.zeros((B, M, 5), known.dtype)], axis=2)
    w0a = W0[:, :C2]
    w0b = W0[:, C2:]
    col = lambda v: v.reshape(-1, 1)

    def off(p, t):
        return jnp.where(p == 0, t // tpb, 0), 0, jnp.where(p == 0, t % tpb, 0)

    def bcast(p, t):
        return jnp.where(p == 0, t // tpb, 0), 0, 0

    grid = (3, nt)
    out = pl.pallas_call(
        functools.partial(_body, bn=bn),
        grid=grid,
        in_specs=[
            pl.BlockSpec((1, 8, tn), off),
            pl.BlockSpec((1, M, 8), bcast),
            pl.BlockSpec((1, C2, M), bcast),
            pl.BlockSpec((1, C1, tn), off),
            pl.BlockSpec((K0, C2), lambda p, t: (0, 0)),
            pl.BlockSpec((K0, C1), lambda p, t: (0, 0)),
            pl.BlockSpec((K1, K0), lambda p, t: (0, 0)),
        ] + [pl.BlockSpec((K0, 1), lambda p, t: (0, 0))] * 3
          + [pl.BlockSpec((K1, 1), lambda p, t: (0, 0))] * 3,
        out_specs=pl.BlockSpec(
            (1, K1, tn),
            lambda p, t: (jnp.where(p == 2, t // tpb, 0), 0,
                          jnp.where(p == 2, t % tpb, 0))),
        out_shape=jax.ShapeDtypeStruct((B, K1, N), jnp.float32),
        scratch_shapes=[
            pltpu.VMEM((nt, K0, tn), jnp.float32),
            pltpu.VMEM((nt, K1, tn), jnp.float32),
            pltpu.VMEM((K0, 8), jnp.float32),
        ],
        compiler_params=pltpu.CompilerParams(
            dimension_semantics=("arbitrary", "arbitrary"),
            vmem_limit_bytes=100 * 1024 * 1024,
        ),
    )(uT8, k8, known_feats, unknow_feats, w0a, w0b, W1,
      col(b0), col(gamma0), col(beta0), col(b1), col(gamma1), col(beta1))
    return out
